# baseline (speedup 1.0000x reference)
import functools

import jax
import jax.numpy as jnp
from jax import lax
from jax.experimental import pallas as pl
from jax.experimental.pallas import tpu as pltpu
from jax.experimental.pallas import tpu_sc as plsc

N = 10000
E = 320000
D = 128

NUM_CORES = 2
NUM_SUBCORES = 16
NW = NUM_CORES * NUM_SUBCORES
CHUNK = 128
CHUNKS_PER_TILE = (E + NW * CHUNK - 1) // (NW * CHUNK)
CHUNKS_PAD = CHUNKS_PER_TILE + 1
EDGES_PAD = NW * CHUNKS_PER_TILE * CHUNK
DUMMY = N
R_ROWS = N + 16
DEG_W = 16

_MESH = plsc.VectorSubcoreMesh(core_axis_name="c", subcore_axis_name="s")


def _striped(s, total, copy_fn):
    r1 = -(-(-(-total // NUM_SUBCORES)) // 8) * 8
    last = total - (NUM_SUBCORES - 1) * r1

    @pl.when(s < NUM_SUBCORES - 1)
    def _():
        copy_fn(s * r1, r1)

    @pl.when(s == NUM_SUBCORES - 1)
    def _():
        copy_fn((NUM_SUBCORES - 1) * r1, last)


NBUF = 2
PHASE = 40
IDX_STAGE = 48
IDX_ROWS = 88


def _edge_body(q_hbm, z_hbm, src_hbm, dst_hbm, ra_hbm, rb_hbm,
               src_v, dst_v, msg0, msg1, r_sh, sg0, sg1):
    c = lax.axis_index("c")
    s = lax.axis_index("s")
    wid = c * NUM_SUBCORES + s

    @pl.when(c == 0)
    def _():
        _striped(s, N, lambda o, n: pltpu.sync_copy(
            q_hbm.at[pl.ds(o, n)], r_sh.at[pl.ds(o, n)]))

    @pl.when(c == 1)
    def _():
        _striped(s, N, lambda o, n: pltpu.sync_copy(
            z_hbm.at[pl.ds(o, n)], r_sh.at[pl.ds(o, n)]))

    @pl.when(s == 0)
    def _():
        pltpu.sync_copy(z_hbm.at[pl.ds(0, R_ROWS - N)],
                        r_sh.at[pl.ds(N, R_ROWS - N)])

    plsc.subcore_barrier()

    msgs = [msg0, msg1]
    sems = [sg0, sg1]

    for base in (0, PHASE):
        pltpu.sync_copy(src_hbm.at[wid, pl.ds(base, IDX_STAGE)], src_v)
        pltpu.sync_copy(dst_hbm.at[wid, pl.ds(base, IDX_STAGE)], dst_v)
        for b in range(NBUF):
            pltpu.async_copy(q_hbm.at[src_v.at[b]], msgs[b], sems[b])

        def pair(i, carry):
            j0 = i * 2
            for b in range(NBUF):
                j = j0 + b
                pltpu.make_async_copy(
                    q_hbm.at[src_v.at[j]], msgs[b], sems[b]).wait()
                pltpu.sync_copy(msgs[b], r_sh.at[dst_v.at[j]], add=True)
                pltpu.async_copy(
                    q_hbm.at[src_v.at[j + NBUF]], msgs[b], sems[b])
            return carry

        lax.fori_loop(0, PHASE // 2, pair, 0)
        for b in range(NBUF):
            pltpu.make_async_copy(
                q_hbm.at[src_v.at[PHASE + b]], msgs[b], sems[b]).wait()
    plsc.subcore_barrier()

    @pl.when(c == 0)
    def _():
        _striped(s, N, lambda o, n: pltpu.sync_copy(
            r_sh.at[pl.ds(o, n)], ra_hbm.at[pl.ds(o, n)]))

    @pl.when(c == 1)
    def _():
        _striped(s, N, lambda o, n: pltpu.sync_copy(
            r_sh.at[pl.ds(o, n)], rb_hbm.at[pl.ds(o, n)]))


_edge_call = pl.kernel(
    _edge_body,
    out_type=[jax.ShapeDtypeStruct((N, D), jnp.float32)] * 2,
    mesh=_MESH,
    scratch_types=[
        pltpu.VMEM((IDX_STAGE, CHUNK), jnp.int32),
        pltpu.VMEM((IDX_STAGE, CHUNK), jnp.int32),
        pltpu.VMEM((CHUNK, D), jnp.float32),
        pltpu.VMEM((CHUNK, D), jnp.float32),
        pltpu.VMEM_SHARED((R_ROWS, D), jnp.float32),
        pltpu.SemaphoreType.DMA,
        pltpu.SemaphoreType.DMA,
    ],
)


def _dinv(dega_ref, degb_ref):
    deg = dega_ref[:, 0:1] + degb_ref[:, 0:1]
    return lax.rsqrt(deg)


def _mm_t(a, w):
    return lax.dot_general(a, w, (((1,), (1,)), ((), ())),
                           preferred_element_type=jnp.float32)


def _tc0_body(x_ref, w_ref, dega_ref, degb_ref, q_ref):
    q_ref[...] = _mm_t(x_ref[...], w_ref[...]) * _dinv(dega_ref, degb_ref)


_tc0 = pl.pallas_call(
    _tc0_body,
    out_shape=jax.ShapeDtypeStruct((N, D), jnp.float32),
)


def _tc_mid_body(ra_ref, rb_ref, dega_ref, degb_ref, b_ref, w_ref, q_ref):
    dinv = _dinv(dega_ref, degb_ref)
    h = jnp.maximum(dinv * (ra_ref[...] + rb_ref[...]) + b_ref[...], 0.0)
    q_ref[...] = _mm_t(h, w_ref[...]) * dinv


_tc_mid = pl.pallas_call(
    _tc_mid_body,
    out_shape=jax.ShapeDtypeStruct((N, D), jnp.float32),
)


def _sigmoid(x):
    return 1.0 / (1.0 + jnp.exp(-x))


def _tc_final_body(ra_ref, rb_ref, dega_ref, degb_ref, b3_ref,
                   ws1_ref, bs1_ref, ws2_ref, bs2_ref,
                   wi1_ref, bi1_ref, wi2_ref, bi2_ref,
                   score_ref, issues_ref):
    dinv = _dinv(dega_ref, degb_ref)
    h = jnp.maximum(dinv * (ra_ref[...] + rb_ref[...]) + b3_ref[...], 0.0)
    g = jnp.sum(h, axis=0, keepdims=True) * (1.0 / N)
    t = jnp.maximum(_mm_t(g, ws1_ref[...]) + bs1_ref[...], 0.0)
    score_ref[...] = _sigmoid(
        jnp.sum(t * ws2_ref[...], axis=1, keepdims=True) + bs2_ref[...])
    u = jnp.maximum(_mm_t(g, wi1_ref[...]) + bi1_ref[...], 0.0)
    issues_ref[...] = _sigmoid(_mm_t(u, wi2_ref[...]) + bi2_ref[...])


_tc_final = pl.pallas_call(
    _tc_final_body,
    out_shape=[jax.ShapeDtypeStruct((1, 1), jnp.float32),
               jax.ShapeDtypeStruct((1, 10), jnp.float32)],
)


def kernel(x, edge_index, W1, b1, W2, b2, W3, b3,
           Ws1, bs1, Ws2, bs2, Wi1, bi1, Wi2, bi2):
    src = edge_index[0]
    dst = edge_index[1]
    pad = EDGES_PAD - E
    src3 = jnp.concatenate(
        [src, jnp.zeros((pad,), jnp.int32)]).reshape(NW, CHUNKS_PER_TILE, CHUNK)
    dst3 = jnp.concatenate(
        [dst, jnp.full((pad,), DUMMY, jnp.int32)]).reshape(NW, CHUNKS_PER_TILE, CHUNK)
    extra = IDX_ROWS - CHUNKS_PER_TILE
    src3 = jnp.concatenate(
        [src3, jnp.zeros((NW, extra, CHUNK), jnp.int32)], axis=1)
    dst3 = jnp.concatenate(
        [dst3, jnp.full((NW, extra, CHUNK), DUMMY, jnp.int32)], axis=1)
    z = jnp.zeros((N, D), jnp.float32)

    ra0, rb0 = _edge_call(jnp.ones((N, D), jnp.float32), z, src3, dst3)
    dega = ra0[:, :8]
    degb = rb0[:, :8]
    q = _tc0(x, W1, dega, degb)
    ra, rb = _edge_call(q, z, src3, dst3)
    q = _tc_mid(ra, rb, dega, degb, b1.reshape(1, D), W2)
    ra, rb = _edge_call(q, z, src3, dst3)
    q = _tc_mid(ra, rb, dega, degb, b2.reshape(1, D), W3)
    ra, rb = _edge_call(q, z, src3, dst3)
    score, issues = _tc_final(
        ra, rb, dega, degb, b3.reshape(1, D),
        Ws1, bs1.reshape(1, -1), Ws2, bs2.reshape(1, -1),
        Wi1, bi1.reshape(1, -1), Wi2, bi2.reshape(1, -1))
    return (score, issues)


# speedup vs baseline: 1.6481x; 1.6481x over previous
import functools

import jax
import jax.numpy as jnp
from jax import lax
from jax.experimental import pallas as pl
from jax.experimental.pallas import tpu as pltpu
from jax.experimental.pallas import tpu_sc as plsc

N = 10000
E = 320000
D = 128

NUM_CORES = 2
NUM_SUBCORES = 16
NW = NUM_CORES * NUM_SUBCORES
CHUNK = 128
CHUNKS_PER_TILE = (E + NW * CHUNK - 1) // (NW * CHUNK)
CHUNKS_PAD = CHUNKS_PER_TILE + 1
EDGES_PAD = NW * CHUNKS_PER_TILE * CHUNK
DUMMY = N
R_ROWS = N + 16
DEG_W = 16

_MESH = plsc.VectorSubcoreMesh(core_axis_name="c", subcore_axis_name="s")


def _striped(s, total, copy_fn):
    r1 = -(-(-(-total // NUM_SUBCORES)) // 8) * 8
    last = total - (NUM_SUBCORES - 1) * r1

    @pl.when(s < NUM_SUBCORES - 1)
    def _():
        copy_fn(s * r1, r1)

    @pl.when(s == NUM_SUBCORES - 1)
    def _():
        copy_fn((NUM_SUBCORES - 1) * r1, last)


PHASE = 40
IDX_STAGE = PHASE
IDX_ROWS = 80


def _edge_body(q_hbm, z_hbm, src_hbm, dst_hbm, ra_hbm, rb_hbm,
               src_v, dst_v, msg0, msg1, r_sh, sg0, sg1, ss0, ss1):
    c = lax.axis_index("c")
    s = lax.axis_index("s")
    wid = c * NUM_SUBCORES + s

    @pl.when(c == 0)
    def _():
        _striped(s, N, lambda o, n: pltpu.sync_copy(
            q_hbm.at[pl.ds(o, n)], r_sh.at[pl.ds(o, n)]))

    @pl.when(c == 1)
    def _():
        _striped(s, N, lambda o, n: pltpu.sync_copy(
            z_hbm.at[pl.ds(o, n)], r_sh.at[pl.ds(o, n)]))

    @pl.when(s == 0)
    def _():
        pltpu.sync_copy(z_hbm.at[pl.ds(0, R_ROWS - N)],
                        r_sh.at[pl.ds(N, R_ROWS - N)])

    plsc.subcore_barrier()

    msgs = [msg0, msg1]
    sg = [sg0, sg1]
    ss = [ss0, ss1]

    def g_start(j, b):
        pltpu.async_copy(q_hbm.at[src_v.at[j]], msgs[b], sg[b])

    def g_wait(j, b):
        pltpu.make_async_copy(q_hbm.at[src_v.at[j]], msgs[b], sg[b]).wait()

    def s_start(j, b):
        pltpu.async_copy(msgs[b], r_sh.at[dst_v.at[j]], ss[b], add=True)

    def s_wait(j, b):
        pltpu.make_async_copy(msgs[b], r_sh.at[dst_v.at[j]], ss[b]).wait()

    def chunk(j, b, first=False):
        g_wait(j, b)
        s_start(j, b)
        if not first:
            s_wait(j - 1, 1 - b)
        g_start(j + 1, 1 - b)

    for base in (0, PHASE):
        pltpu.sync_copy(src_hbm.at[wid, pl.ds(base, IDX_STAGE)], src_v)
        pltpu.sync_copy(dst_hbm.at[wid, pl.ds(base, IDX_STAGE)], dst_v)
        g_start(0, 0)
        chunk(0, 0, first=True)

        def pair(i, carry):
            j = 2 * i + 1
            chunk(j, 1)
            chunk(j + 1, 0)
            return carry

        lax.fori_loop(0, (PHASE - 2) // 2, pair, 0)
        g_wait(PHASE - 1, 1)
        s_start(PHASE - 1, 1)
        s_wait(PHASE - 2, 0)
        s_wait(PHASE - 1, 1)
    plsc.subcore_barrier()

    @pl.when(c == 0)
    def _():
        _striped(s, N, lambda o, n: pltpu.sync_copy(
            r_sh.at[pl.ds(o, n)], ra_hbm.at[pl.ds(o, n)]))

    @pl.when(c == 1)
    def _():
        _striped(s, N, lambda o, n: pltpu.sync_copy(
            r_sh.at[pl.ds(o, n)], rb_hbm.at[pl.ds(o, n)]))


_edge_call = pl.kernel(
    _edge_body,
    out_type=[jax.ShapeDtypeStruct((N, D), jnp.float32)] * 2,
    mesh=_MESH,
    scratch_types=[
        pltpu.VMEM((IDX_STAGE, CHUNK), jnp.int32),
        pltpu.VMEM((IDX_STAGE, CHUNK), jnp.int32),
        pltpu.VMEM((CHUNK, D), jnp.float32),
        pltpu.VMEM((CHUNK, D), jnp.float32),
        pltpu.VMEM_SHARED((R_ROWS, D), jnp.float32),
        pltpu.SemaphoreType.DMA,
        pltpu.SemaphoreType.DMA,
        pltpu.SemaphoreType.DMA,
        pltpu.SemaphoreType.DMA,
    ],
)


def _dinv(dega_ref, degb_ref):
    deg = dega_ref[:, 0:1] + degb_ref[:, 0:1]
    return lax.rsqrt(deg)


def _mm_t(a, w):
    return lax.dot_general(a, w, (((1,), (1,)), ((), ())),
                           preferred_element_type=jnp.float32)


def _tc0_body(x_ref, w_ref, dega_ref, degb_ref, q_ref):
    q_ref[...] = _mm_t(x_ref[...], w_ref[...]) * _dinv(dega_ref, degb_ref)


_tc0 = pl.pallas_call(
    _tc0_body,
    out_shape=jax.ShapeDtypeStruct((N, D), jnp.float32),
)


def _tc_mid_body(ra_ref, rb_ref, dega_ref, degb_ref, b_ref, w_ref, q_ref):
    dinv = _dinv(dega_ref, degb_ref)
    h = jnp.maximum(dinv * (ra_ref[...] + rb_ref[...]) + b_ref[...], 0.0)
    q_ref[...] = _mm_t(h, w_ref[...]) * dinv


_tc_mid = pl.pallas_call(
    _tc_mid_body,
    out_shape=jax.ShapeDtypeStruct((N, D), jnp.float32),
)


def _sigmoid(x):
    return 1.0 / (1.0 + jnp.exp(-x))


def _tc_final_body(ra_ref, rb_ref, dega_ref, degb_ref, b3_ref,
                   ws1_ref, bs1_ref, ws2_ref, bs2_ref,
                   wi1_ref, bi1_ref, wi2_ref, bi2_ref,
                   score_ref, issues_ref):
    dinv = _dinv(dega_ref, degb_ref)
    h = jnp.maximum(dinv * (ra_ref[...] + rb_ref[...]) + b3_ref[...], 0.0)
    g = jnp.sum(h, axis=0, keepdims=True) * (1.0 / N)
    t = jnp.maximum(_mm_t(g, ws1_ref[...]) + bs1_ref[...], 0.0)
    score_ref[...] = _sigmoid(
        jnp.sum(t * ws2_ref[...], axis=1, keepdims=True) + bs2_ref[...])
    u = jnp.maximum(_mm_t(g, wi1_ref[...]) + bi1_ref[...], 0.0)
    issues_ref[...] = _sigmoid(_mm_t(u, wi2_ref[...]) + bi2_ref[...])


_tc_final = pl.pallas_call(
    _tc_final_body,
    out_shape=[jax.ShapeDtypeStruct((1, 1), jnp.float32),
               jax.ShapeDtypeStruct((1, 10), jnp.float32)],
)


def kernel(x, edge_index, W1, b1, W2, b2, W3, b3,
           Ws1, bs1, Ws2, bs2, Wi1, bi1, Wi2, bi2):
    src = edge_index[0]
    dst = edge_index[1]
    pad = EDGES_PAD - E
    src3 = jnp.concatenate(
        [src, jnp.zeros((pad,), jnp.int32)]).reshape(NW, CHUNKS_PER_TILE, CHUNK)
    dst3 = jnp.concatenate(
        [dst, jnp.full((pad,), DUMMY, jnp.int32)]).reshape(NW, CHUNKS_PER_TILE, CHUNK)
    extra = IDX_ROWS - CHUNKS_PER_TILE
    src3 = jnp.concatenate(
        [src3, jnp.zeros((NW, extra, CHUNK), jnp.int32)], axis=1)
    dst3 = jnp.concatenate(
        [dst3, jnp.full((NW, extra, CHUNK), DUMMY, jnp.int32)], axis=1)
    z = jnp.zeros((N, D), jnp.float32)

    ra0, rb0 = _edge_call(jnp.ones((N, D), jnp.float32), z, src3, dst3)
    dega = ra0[:, :8]
    degb = rb0[:, :8]
    q = _tc0(x, W1, dega, degb)
    ra, rb = _edge_call(q, z, src3, dst3)
    q = _tc_mid(ra, rb, dega, degb, b1.reshape(1, D), W2)
    ra, rb = _edge_call(q, z, src3, dst3)
    q = _tc_mid(ra, rb, dega, degb, b2.reshape(1, D), W3)
    ra, rb = _edge_call(q, z, src3, dst3)
    score, issues = _tc_final(
        ra, rb, dega, degb, b3.reshape(1, D),
        Ws1, bs1.reshape(1, -1), Ws2, bs2.reshape(1, -1),
        Wi1, bi1.reshape(1, -1), Wi2, bi2.reshape(1, -1))
    return (score, issues)


# speedup vs baseline: 2.3728x; 1.4397x over previous
import functools

import jax
import jax.numpy as jnp
from jax import lax
from jax.experimental import pallas as pl
from jax.experimental.pallas import tpu as pltpu
from jax.experimental.pallas import tpu_sc as plsc

N = 10000
E = 320000
D = 128

NUM_CORES = 2
NUM_SUBCORES = 16
NW = NUM_CORES * NUM_SUBCORES
CHUNK = 128
CHUNKS_PER_TILE = (E + NW * CHUNK - 1) // (NW * CHUNK)
CHUNKS_PAD = CHUNKS_PER_TILE + 1
EDGES_PAD = NW * CHUNKS_PER_TILE * CHUNK
DUMMY = N
R_ROWS = N + 16
DEG_W = 16

_MESH = plsc.VectorSubcoreMesh(core_axis_name="c", subcore_axis_name="s")


def _striped(s, total, copy_fn, align=8):
    r1 = -(-(-(-total // NUM_SUBCORES)) // align) * align
    last = total - (NUM_SUBCORES - 1) * r1

    @pl.when(s < NUM_SUBCORES - 1)
    def _():
        copy_fn(s * r1, r1)

    @pl.when(s == NUM_SUBCORES - 1)
    def _():
        copy_fn((NUM_SUBCORES - 1) * r1, last)


IDX_ROWS = 80
T_C0 = 54
T_C1 = 104
R_BAL = max(T_C0, T_C1)


def _edge_body(q_hbm, z_hbm, src_hbm, dst_hbm, ra_hbm, rb_hbm,
               src_v, dst_v, msg_v, r_sh, sem):
    c = lax.axis_index("c")
    s = lax.axis_index("s")
    wid = c * NUM_SUBCORES + s

    @pl.when(c == 0)
    def _():
        _striped(s, N, lambda o, n: pltpu.sync_copy(
            q_hbm.at[pl.ds(o, n)], r_sh.at[pl.ds(o, n)]))

    @pl.when(c == 1)
    def _():
        _striped(s, N, lambda o, n: pltpu.sync_copy(
            z_hbm.at[pl.ds(o, n)], r_sh.at[pl.ds(o, n)]))

    @pl.when(s == 0)
    def _():
        pltpu.sync_copy(z_hbm.at[pl.ds(0, R_ROWS - N)],
                        r_sh.at[pl.ds(N, R_ROWS - N)])

    plsc.subcore_barrier()

    pltpu.sync_copy(src_hbm.at[wid], src_v)
    pltpu.sync_copy(dst_hbm.at[wid], dst_v)

    def chunk(j, carry):
        pltpu.async_copy(q_hbm.at[src_v.at[j]], msg_v, sem).wait()
        pltpu.sync_copy(msg_v, r_sh.at[dst_v.at[j]], add=True)
        return carry

    @pl.when(c == 0)
    def _():
        lax.fori_loop(0, T_C0, chunk, 0)

    @pl.when(c == 1)
    def _():
        lax.fori_loop(0, T_C1, chunk, 0)

    plsc.subcore_barrier()

    @pl.when(c == 0)
    def _():
        _striped(s, N, lambda o, n: pltpu.sync_copy(
            r_sh.at[pl.ds(o, n)], ra_hbm.at[pl.ds(o, n)]))

    @pl.when(c == 1)
    def _():
        _striped(s, N, lambda o, n: pltpu.sync_copy(
            r_sh.at[pl.ds(o, n)], rb_hbm.at[pl.ds(o, n)]))


_edge_call = pl.kernel(
    _edge_body,
    out_type=[jax.ShapeDtypeStruct((N, D), jnp.float32)] * 2,
    mesh=_MESH,
    scratch_types=[
        pltpu.VMEM((R_BAL, CHUNK), jnp.int32),
        pltpu.VMEM((R_BAL, CHUNK), jnp.int32),
        pltpu.VMEM((CHUNK, D), jnp.float32),
        pltpu.VMEM_SHARED((R_ROWS, D), jnp.float32),
        pltpu.SemaphoreType.DMA,
    ],
)


def _deg_body(ones_hbm, z_hbm, dst_hbm, dega_hbm, degb_hbm,
              dst_v, ones_v, deg_sh):
    c = lax.axis_index("c")
    s = lax.axis_index("s")
    wid = c * NUM_SUBCORES + s
    pltpu.sync_copy(dst_hbm.at[wid], dst_v)
    pltpu.sync_copy(ones_hbm, ones_v)
    _striped(s, N, lambda o, n: pltpu.sync_copy(
        z_hbm.at[pl.ds(o, n)], deg_sh.at[pl.ds(o, n)]))

    @pl.when(s == 0)
    def _():
        pltpu.sync_copy(z_hbm.at[pl.ds(0, R_ROWS - N)],
                        deg_sh.at[pl.ds(N, R_ROWS - N)])

    plsc.subcore_barrier()

    def chunk(j, carry):
        pltpu.sync_copy(ones_v, deg_sh.at[dst_v.at[j]], add=True)
        return carry

    lax.fori_loop(0, CHUNKS_PER_TILE, chunk, 0)
    plsc.subcore_barrier()

    @pl.when(c == 0)
    def _():
        _striped(s, N, lambda o, n: pltpu.sync_copy(
            deg_sh.at[pl.ds(o, n)], dega_hbm.at[pl.ds(o, n)]))

    @pl.when(c == 1)
    def _():
        _striped(s, N, lambda o, n: pltpu.sync_copy(
            deg_sh.at[pl.ds(o, n)], degb_hbm.at[pl.ds(o, n)]))


_deg_call = pl.kernel(
    _deg_body,
    out_type=[jax.ShapeDtypeStruct((N, D), jnp.float32)] * 2,
    mesh=_MESH,
    scratch_types=[
        pltpu.VMEM((IDX_ROWS, CHUNK), jnp.int32),
        pltpu.VMEM((CHUNK, D), jnp.float32),
        pltpu.VMEM_SHARED((R_ROWS, D), jnp.float32),
    ],
)


def _dinv(dega_ref, degb_ref):
    deg = dega_ref[:, 0:1] + degb_ref[:, 0:1] + 1.0
    return lax.rsqrt(deg)


def _mm_t(a, w):
    return lax.dot_general(a, w, (((1,), (1,)), ((), ())),
                           preferred_element_type=jnp.float32)


def _tc0_body(x_ref, w_ref, dega_ref, degb_ref, q_ref):
    q_ref[...] = _mm_t(x_ref[...], w_ref[...]) * _dinv(dega_ref, degb_ref)


_tc0 = pl.pallas_call(
    _tc0_body,
    out_shape=jax.ShapeDtypeStruct((N, D), jnp.float32),
)


def _tc_mid_body(ra_ref, rb_ref, dega_ref, degb_ref, b_ref, w_ref, q_ref):
    dinv = _dinv(dega_ref, degb_ref)
    h = jnp.maximum(dinv * (ra_ref[...] + rb_ref[...]) + b_ref[...], 0.0)
    q_ref[...] = _mm_t(h, w_ref[...]) * dinv


_tc_mid = pl.pallas_call(
    _tc_mid_body,
    out_shape=jax.ShapeDtypeStruct((N, D), jnp.float32),
)


def _sigmoid(x):
    return 1.0 / (1.0 + jnp.exp(-x))


def _tc_final_body(ra_ref, rb_ref, dega_ref, degb_ref, b3_ref,
                   ws1_ref, bs1_ref, ws2_ref, bs2_ref,
                   wi1_ref, bi1_ref, wi2_ref, bi2_ref,
                   score_ref, issues_ref):
    dinv = _dinv(dega_ref, degb_ref)
    h = jnp.maximum(dinv * (ra_ref[...] + rb_ref[...]) + b3_ref[...], 0.0)
    g = jnp.sum(h, axis=0, keepdims=True) * (1.0 / N)
    t = jnp.maximum(_mm_t(g, ws1_ref[...]) + bs1_ref[...], 0.0)
    score_ref[...] = _sigmoid(
        jnp.sum(t * ws2_ref[...], axis=1, keepdims=True) + bs2_ref[...])
    u = jnp.maximum(_mm_t(g, wi1_ref[...]) + bi1_ref[...], 0.0)
    issues_ref[...] = _sigmoid(_mm_t(u, wi2_ref[...]) + bi2_ref[...])


_tc_final = pl.pallas_call(
    _tc_final_body,
    out_shape=[jax.ShapeDtypeStruct((1, 1), jnp.float32),
               jax.ShapeDtypeStruct((1, 10), jnp.float32)],
)


def kernel(x, edge_index, W1, b1, W2, b2, W3, b3,
           Ws1, bs1, Ws2, bs2, Wi1, bi1, Wi2, bi2):
    src = edge_index[0]
    dst = edge_index[1]
    pad = EDGES_PAD - E
    src_p = jnp.concatenate([src, jnp.zeros((pad,), jnp.int32)])
    dst_p = jnp.concatenate([dst, jnp.full((pad,), DUMMY, jnp.int32)])

    dst3_sym = dst_p.reshape(NW, CHUNKS_PER_TILE, CHUNK)
    dst3_sym = jnp.concatenate(
        [dst3_sym,
         jnp.full((NW, IDX_ROWS - CHUNKS_PER_TILE, CHUNK), DUMMY, jnp.int32)],
        axis=1)

    cut = NUM_SUBCORES * T_C0 * CHUNK

    def _bal(flat, fill):
        a = flat[:cut].reshape(NUM_SUBCORES, T_C0, CHUNK)
        a = jnp.concatenate(
            [a, jnp.full((NUM_SUBCORES, R_BAL - T_C0, CHUNK), fill,
                         jnp.int32)], axis=1)
        b = flat[cut:].reshape(NUM_SUBCORES, T_C1, CHUNK)
        b = jnp.concatenate(
            [b, jnp.full((NUM_SUBCORES, R_BAL - T_C1, CHUNK), fill,
                         jnp.int32)], axis=1)
        return jnp.concatenate([a, b], axis=0)

    src3 = _bal(src_p, 0)
    dst3 = _bal(dst_p, DUMMY)
    z = jnp.zeros((N, D), jnp.float32)
    ones_chunk = jnp.ones((CHUNK, D), jnp.float32)

    dega, degb = _deg_call(ones_chunk, z, dst3_sym)
    dega = dega[:, :8]
    degb = degb[:, :8]
    q = _tc0(x, W1, dega, degb)
    ra, rb = _edge_call(q, z, src3, dst3)
    q = _tc_mid(ra, rb, dega, degb, b1.reshape(1, D), W2)
    ra, rb = _edge_call(q, z, src3, dst3)
    q = _tc_mid(ra, rb, dega, degb, b2.reshape(1, D), W3)
    ra, rb = _edge_call(q, z, src3, dst3)
    score, issues = _tc_final(
        ra, rb, dega, degb, b3.reshape(1, D),
        Ws1, bs1.reshape(1, -1), Ws2, bs2.reshape(1, -1),
        Wi1, bi1.reshape(1, -1), Wi2, bi2.reshape(1, -1))
    return (score, issues)


# speedup vs baseline: 2.9767x; 1.2545x over previous
import functools

import jax
import jax.numpy as jnp
from jax import lax
from jax.experimental import pallas as pl
from jax.experimental.pallas import tpu as pltpu
from jax.experimental.pallas import tpu_sc as plsc

N = 10000
E = 320000
D = 128

NUM_CORES = 2
NUM_SUBCORES = 16
NW = NUM_CORES * NUM_SUBCORES
CHUNK = 128
CHUNKS_PER_TILE = (E + NW * CHUNK - 1) // (NW * CHUNK)
CHUNKS_PAD = CHUNKS_PER_TILE + 1
EDGES_PAD = NW * CHUNKS_PER_TILE * CHUNK
DUMMY = N
R_ROWS = N + 16
DEG_W = 16

_MESH = plsc.VectorSubcoreMesh(core_axis_name="c", subcore_axis_name="s")


def _striped(s, total, copy_fn, align=8):
    r1 = -(-(-(-total // NUM_SUBCORES)) // align) * align
    last = total - (NUM_SUBCORES - 1) * r1

    @pl.when(s < NUM_SUBCORES - 1)
    def _():
        copy_fn(s * r1, r1)

    @pl.when(s == NUM_SUBCORES - 1)
    def _():
        copy_fn((NUM_SUBCORES - 1) * r1, last)


IDX_ROWS = 80
T_C0 = 104
T_C1 = 54
R_BAL = max(T_C0, T_C1)


def _edge_body(q_hbm, z_hbm, src_hbm, dst_hbm, ra_hbm, rb_hbm,
               src_v, dst_v, msg_v, r_sh, sem):
    c = lax.axis_index("c")
    s = lax.axis_index("s")
    wid = c * NUM_SUBCORES + s

    @pl.when(c == 0)
    def _():
        _striped(s, N, lambda o, n: pltpu.sync_copy(
            q_hbm.at[pl.ds(o, n)], r_sh.at[pl.ds(o, n)]))

    @pl.when(c == 1)
    def _():
        _striped(s, N, lambda o, n: pltpu.sync_copy(
            z_hbm.at[pl.ds(o, n)], r_sh.at[pl.ds(o, n)]))

    @pl.when(s == 0)
    def _():
        pltpu.sync_copy(z_hbm.at[pl.ds(0, R_ROWS - N)],
                        r_sh.at[pl.ds(N, R_ROWS - N)])

    plsc.subcore_barrier()

    pltpu.sync_copy(src_hbm.at[wid], src_v)
    pltpu.sync_copy(dst_hbm.at[wid], dst_v)

    def chunk(j, carry):
        pltpu.async_copy(q_hbm.at[src_v.at[j]], msg_v, sem).wait()
        pltpu.sync_copy(msg_v, r_sh.at[dst_v.at[j]], add=True)
        return carry

    @pl.when(c == 0)
    def _():
        lax.fori_loop(0, T_C0, chunk, 0)

    @pl.when(c == 1)
    def _():
        lax.fori_loop(0, T_C1, chunk, 0)

    plsc.subcore_barrier()

    @pl.when(c == 0)
    def _():
        _striped(s, N, lambda o, n: pltpu.sync_copy(
            r_sh.at[pl.ds(o, n)], ra_hbm.at[pl.ds(o, n)]))

    @pl.when(c == 1)
    def _():
        _striped(s, N, lambda o, n: pltpu.sync_copy(
            r_sh.at[pl.ds(o, n)], rb_hbm.at[pl.ds(o, n)]))


_edge_call = pl.kernel(
    _edge_body,
    out_type=[jax.ShapeDtypeStruct((N, D), jnp.float32)] * 2,
    mesh=_MESH,
    scratch_types=[
        pltpu.VMEM((R_BAL, CHUNK), jnp.int32),
        pltpu.VMEM((R_BAL, CHUNK), jnp.int32),
        pltpu.VMEM((CHUNK, D), jnp.float32),
        pltpu.VMEM_SHARED((R_ROWS, D), jnp.float32),
        pltpu.SemaphoreType.DMA,
    ],
)


def _deg_body(ones_hbm, z_hbm, dst_hbm, dega_hbm, degb_hbm,
              dst_v, ones_v, deg_sh):
    c = lax.axis_index("c")
    s = lax.axis_index("s")
    wid = c * NUM_SUBCORES + s
    pltpu.sync_copy(dst_hbm.at[wid], dst_v)
    pltpu.sync_copy(ones_hbm, ones_v)
    _striped(s, N, lambda o, n: pltpu.sync_copy(
        z_hbm.at[pl.ds(o, n)], deg_sh.at[pl.ds(o, n)]))

    @pl.when(s == 0)
    def _():
        pltpu.sync_copy(z_hbm.at[pl.ds(0, R_ROWS - N)],
                        deg_sh.at[pl.ds(N, R_ROWS - N)])

    plsc.subcore_barrier()

    def chunk(j, carry):
        pltpu.sync_copy(ones_v, deg_sh.at[dst_v.at[j]], add=True)
        return carry

    lax.fori_loop(0, CHUNKS_PER_TILE, chunk, 0)
    plsc.subcore_barrier()

    @pl.when(c == 0)
    def _():
        _striped(s, N, lambda o, n: pltpu.sync_copy(
            deg_sh.at[pl.ds(o, n)], dega_hbm.at[pl.ds(o, n)]))

    @pl.when(c == 1)
    def _():
        _striped(s, N, lambda o, n: pltpu.sync_copy(
            deg_sh.at[pl.ds(o, n)], degb_hbm.at[pl.ds(o, n)]))


_deg_call = pl.kernel(
    _deg_body,
    out_type=[jax.ShapeDtypeStruct((N, D), jnp.float32)] * 2,
    mesh=_MESH,
    scratch_types=[
        pltpu.VMEM((IDX_ROWS, CHUNK), jnp.int32),
        pltpu.VMEM((CHUNK, D), jnp.float32),
        pltpu.VMEM_SHARED((R_ROWS, D), jnp.float32),
    ],
)


def _dinv(dega_ref, degb_ref):
    deg = dega_ref[:, 0:1] + degb_ref[:, 0:1] + 1.0
    return lax.rsqrt(deg)


def _mm_t(a, w):
    return lax.dot_general(a, w, (((1,), (1,)), ((), ())),
                           preferred_element_type=jnp.float32)


def _tc0_body(x_ref, w_ref, dega_ref, degb_ref, q_ref):
    q_ref[...] = _mm_t(x_ref[...], w_ref[...]) * _dinv(dega_ref, degb_ref)


_tc0 = pl.pallas_call(
    _tc0_body,
    out_shape=jax.ShapeDtypeStruct((N, D), jnp.float32),
)


def _tc_mid_body(ra_ref, rb_ref, dega_ref, degb_ref, b_ref, w_ref, q_ref):
    dinv = _dinv(dega_ref, degb_ref)
    h = jnp.maximum(dinv * (ra_ref[...] + rb_ref[...]) + b_ref[...], 0.0)
    q_ref[...] = _mm_t(h, w_ref[...]) * dinv


_tc_mid = pl.pallas_call(
    _tc_mid_body,
    out_shape=jax.ShapeDtypeStruct((N, D), jnp.float32),
)


def _sigmoid(x):
    return 1.0 / (1.0 + jnp.exp(-x))


def _tc_final_body(ra_ref, rb_ref, dega_ref, degb_ref, b3_ref,
                   ws1_ref, bs1_ref, ws2_ref, bs2_ref,
                   wi1_ref, bi1_ref, wi2_ref, bi2_ref,
                   score_ref, issues_ref):
    dinv = _dinv(dega_ref, degb_ref)
    h = jnp.maximum(dinv * (ra_ref[...] + rb_ref[...]) + b3_ref[...], 0.0)
    g = jnp.sum(h, axis=0, keepdims=True) * (1.0 / N)
    t = jnp.maximum(_mm_t(g, ws1_ref[...]) + bs1_ref[...], 0.0)
    score_ref[...] = _sigmoid(
        jnp.sum(t * ws2_ref[...], axis=1, keepdims=True) + bs2_ref[...])
    u = jnp.maximum(_mm_t(g, wi1_ref[...]) + bi1_ref[...], 0.0)
    issues_ref[...] = _sigmoid(_mm_t(u, wi2_ref[...]) + bi2_ref[...])


_tc_final = pl.pallas_call(
    _tc_final_body,
    out_shape=[jax.ShapeDtypeStruct((1, 1), jnp.float32),
               jax.ShapeDtypeStruct((1, 10), jnp.float32)],
)


def kernel(x, edge_index, W1, b1, W2, b2, W3, b3,
           Ws1, bs1, Ws2, bs2, Wi1, bi1, Wi2, bi2):
    src = edge_index[0]
    dst = edge_index[1]
    pad = EDGES_PAD - E
    src_p = jnp.concatenate([src, jnp.zeros((pad,), jnp.int32)])
    dst_p = jnp.concatenate([dst, jnp.full((pad,), DUMMY, jnp.int32)])

    dst3_sym = dst_p.reshape(NW, CHUNKS_PER_TILE, CHUNK)
    dst3_sym = jnp.concatenate(
        [dst3_sym,
         jnp.full((NW, IDX_ROWS - CHUNKS_PER_TILE, CHUNK), DUMMY, jnp.int32)],
        axis=1)

    cut = NUM_SUBCORES * T_C0 * CHUNK

    def _bal(flat, fill):
        a = flat[:cut].reshape(NUM_SUBCORES, T_C0, CHUNK)
        a = jnp.concatenate(
            [a, jnp.full((NUM_SUBCORES, R_BAL - T_C0, CHUNK), fill,
                         jnp.int32)], axis=1)
        b = flat[cut:].reshape(NUM_SUBCORES, T_C1, CHUNK)
        b = jnp.concatenate(
            [b, jnp.full((NUM_SUBCORES, R_BAL - T_C1, CHUNK), fill,
                         jnp.int32)], axis=1)
        return jnp.concatenate([a, b], axis=0)

    src3 = _bal(src_p, 0)
    dst3 = _bal(dst_p, DUMMY)
    z = jnp.zeros((N, D), jnp.float32)
    ones_chunk = jnp.ones((CHUNK, D), jnp.float32)

    dega, degb = _deg_call(ones_chunk, z, dst3_sym)
    dega = dega[:, :8]
    degb = degb[:, :8]
    q = _tc0(x, W1, dega, degb)
    ra, rb = _edge_call(q, z, src3, dst3)
    q = _tc_mid(ra, rb, dega, degb, b1.reshape(1, D), W2)
    ra, rb = _edge_call(q, z, src3, dst3)
    q = _tc_mid(ra, rb, dega, degb, b2.reshape(1, D), W3)
    ra, rb = _edge_call(q, z, src3, dst3)
    score, issues = _tc_final(
        ra, rb, dega, degb, b3.reshape(1, D),
        Ws1, bs1.reshape(1, -1), Ws2, bs2.reshape(1, -1),
        Wi1, bi1.reshape(1, -1), Wi2, bi2.reshape(1, -1))
    return (score, issues)


# speedup vs baseline: 3.1094x; 1.0446x over previous
import functools

import jax
import jax.numpy as jnp
from jax import lax
from jax.experimental import pallas as pl
from jax.experimental.pallas import tpu as pltpu
from jax.experimental.pallas import tpu_sc as plsc

N = 10000
E = 320000
D = 128

NUM_CORES = 2
NUM_SUBCORES = 16
NW = NUM_CORES * NUM_SUBCORES
CHUNK = 128
CHUNKS_PER_TILE = (E + NW * CHUNK - 1) // (NW * CHUNK)
CHUNKS_PAD = CHUNKS_PER_TILE + 1
EDGES_PAD = NW * CHUNKS_PER_TILE * CHUNK
DUMMY = N
R_ROWS = N + 16
DEG_W = 16

_MESH = plsc.VectorSubcoreMesh(core_axis_name="c", subcore_axis_name="s")


def _striped(s, total, copy_fn, align=8):
    r1 = -(-(-(-total // NUM_SUBCORES)) // align) * align
    last = total - (NUM_SUBCORES - 1) * r1

    @pl.when(s < NUM_SUBCORES - 1)
    def _():
        copy_fn(s * r1, r1)

    @pl.when(s == NUM_SUBCORES - 1)
    def _():
        copy_fn((NUM_SUBCORES - 1) * r1, last)


IDX_ROWS = 80
T_C0 = 94
T_C1 = 64
R_BAL = max(T_C0, T_C1)


def _edge_body(q_hbm, z_hbm, src_hbm, dst_hbm, ra_hbm, rb_hbm,
               src_v, dst_v, msg_v, r_sh, sem):
    c = lax.axis_index("c")
    s = lax.axis_index("s")
    wid = c * NUM_SUBCORES + s

    @pl.when(c == 0)
    def _():
        _striped(s, N, lambda o, n: pltpu.sync_copy(
            q_hbm.at[pl.ds(o, n)], r_sh.at[pl.ds(o, n)]))

    @pl.when(c == 1)
    def _():
        _striped(s, N, lambda o, n: pltpu.sync_copy(
            z_hbm.at[pl.ds(o, n)], r_sh.at[pl.ds(o, n)]))

    @pl.when(s == 0)
    def _():
        pltpu.sync_copy(z_hbm.at[pl.ds(0, R_ROWS - N)],
                        r_sh.at[pl.ds(N, R_ROWS - N)])

    plsc.subcore_barrier()

    pltpu.sync_copy(src_hbm.at[wid], src_v)
    pltpu.sync_copy(dst_hbm.at[wid], dst_v)

    def chunk(j, carry):
        pltpu.async_copy(q_hbm.at[src_v.at[j]], msg_v, sem).wait()
        pltpu.sync_copy(msg_v, r_sh.at[dst_v.at[j]], add=True)
        return carry

    @pl.when(c == 0)
    def _():
        lax.fori_loop(0, T_C0, chunk, 0)

    @pl.when(c == 1)
    def _():
        lax.fori_loop(0, T_C1, chunk, 0)

    plsc.subcore_barrier()

    @pl.when(c == 0)
    def _():
        _striped(s, N, lambda o, n: pltpu.sync_copy(
            r_sh.at[pl.ds(o, n)], ra_hbm.at[pl.ds(o, n)]))

    @pl.when(c == 1)
    def _():
        _striped(s, N, lambda o, n: pltpu.sync_copy(
            r_sh.at[pl.ds(o, n)], rb_hbm.at[pl.ds(o, n)]))


_edge_call = pl.kernel(
    _edge_body,
    out_type=[jax.ShapeDtypeStruct((N, D), jnp.float32)] * 2,
    mesh=_MESH,
    scratch_types=[
        pltpu.VMEM((R_BAL, CHUNK), jnp.int32),
        pltpu.VMEM((R_BAL, CHUNK), jnp.int32),
        pltpu.VMEM((CHUNK, D), jnp.float32),
        pltpu.VMEM_SHARED((R_ROWS, D), jnp.float32),
        pltpu.SemaphoreType.DMA,
    ],
)


def _deg_body(ones_hbm, z_hbm, dst_hbm, dega_hbm, degb_hbm,
              dst_v, ones_v, deg_sh):
    c = lax.axis_index("c")
    s = lax.axis_index("s")
    wid = c * NUM_SUBCORES + s
    pltpu.sync_copy(dst_hbm.at[wid], dst_v)
    pltpu.sync_copy(ones_hbm, ones_v)
    _striped(s, N, lambda o, n: pltpu.sync_copy(
        z_hbm.at[pl.ds(o, n)], deg_sh.at[pl.ds(o, n)]))

    @pl.when(s == 0)
    def _():
        pltpu.sync_copy(z_hbm.at[pl.ds(0, R_ROWS - N)],
                        deg_sh.at[pl.ds(N, R_ROWS - N)])

    plsc.subcore_barrier()

    def chunk(j, carry):
        pltpu.sync_copy(ones_v, deg_sh.at[dst_v.at[j]], add=True)
        return carry

    lax.fori_loop(0, CHUNKS_PER_TILE, chunk, 0)
    plsc.subcore_barrier()

    @pl.when(c == 0)
    def _():
        _striped(s, N, lambda o, n: pltpu.sync_copy(
            deg_sh.at[pl.ds(o, n)], dega_hbm.at[pl.ds(o, n)]))

    @pl.when(c == 1)
    def _():
        _striped(s, N, lambda o, n: pltpu.sync_copy(
            deg_sh.at[pl.ds(o, n)], degb_hbm.at[pl.ds(o, n)]))


_deg_call = pl.kernel(
    _deg_body,
    out_type=[jax.ShapeDtypeStruct((N, D), jnp.float32)] * 2,
    mesh=_MESH,
    scratch_types=[
        pltpu.VMEM((IDX_ROWS, CHUNK), jnp.int32),
        pltpu.VMEM((CHUNK, D), jnp.float32),
        pltpu.VMEM_SHARED((R_ROWS, D), jnp.float32),
    ],
)


def _dinv(dega_ref, degb_ref):
    deg = dega_ref[:, 0:1] + degb_ref[:, 0:1] + 1.0
    return lax.rsqrt(deg)


def _mm_t(a, w):
    return lax.dot_general(a, w, (((1,), (1,)), ((), ())),
                           preferred_element_type=jnp.float32)


def _tc0_body(x_ref, w_ref, dega_ref, degb_ref, q_ref):
    q_ref[...] = _mm_t(x_ref[...], w_ref[...]) * _dinv(dega_ref, degb_ref)


_tc0 = pl.pallas_call(
    _tc0_body,
    out_shape=jax.ShapeDtypeStruct((N, D), jnp.float32),
)


def _tc_mid_body(ra_ref, rb_ref, dega_ref, degb_ref, b_ref, w_ref, q_ref):
    dinv = _dinv(dega_ref, degb_ref)
    h = jnp.maximum(dinv * (ra_ref[...] + rb_ref[...]) + b_ref[...], 0.0)
    q_ref[...] = _mm_t(h, w_ref[...]) * dinv


_tc_mid = pl.pallas_call(
    _tc_mid_body,
    out_shape=jax.ShapeDtypeStruct((N, D), jnp.float32),
)


def _sigmoid(x):
    return 1.0 / (1.0 + jnp.exp(-x))


def _tc_final_body(ra_ref, rb_ref, dega_ref, degb_ref, b3_ref,
                   ws1_ref, bs1_ref, ws2_ref, bs2_ref,
                   wi1_ref, bi1_ref, wi2_ref, bi2_ref,
                   score_ref, issues_ref):
    dinv = _dinv(dega_ref, degb_ref)
    h = jnp.maximum(dinv * (ra_ref[...] + rb_ref[...]) + b3_ref[...], 0.0)
    g = jnp.sum(h, axis=0, keepdims=True) * (1.0 / N)
    t = jnp.maximum(_mm_t(g, ws1_ref[...]) + bs1_ref[...], 0.0)
    score_ref[...] = _sigmoid(
        jnp.sum(t * ws2_ref[...], axis=1, keepdims=True) + bs2_ref[...])
    u = jnp.maximum(_mm_t(g, wi1_ref[...]) + bi1_ref[...], 0.0)
    issues_ref[...] = _sigmoid(_mm_t(u, wi2_ref[...]) + bi2_ref[...])


_tc_final = pl.pallas_call(
    _tc_final_body,
    out_shape=[jax.ShapeDtypeStruct((1, 1), jnp.float32),
               jax.ShapeDtypeStruct((1, 10), jnp.float32)],
)


def kernel(x, edge_index, W1, b1, W2, b2, W3, b3,
           Ws1, bs1, Ws2, bs2, Wi1, bi1, Wi2, bi2):
    src = edge_index[0]
    dst = edge_index[1]
    pad = EDGES_PAD - E
    src_p = jnp.concatenate([src, jnp.zeros((pad,), jnp.int32)])
    dst_p = jnp.concatenate([dst, jnp.full((pad,), DUMMY, jnp.int32)])

    dst3_sym = dst_p.reshape(NW, CHUNKS_PER_TILE, CHUNK)
    dst3_sym = jnp.concatenate(
        [dst3_sym,
         jnp.full((NW, IDX_ROWS - CHUNKS_PER_TILE, CHUNK), DUMMY, jnp.int32)],
        axis=1)

    cut = NUM_SUBCORES * T_C0 * CHUNK

    def _bal(flat, fill):
        a = flat[:cut].reshape(NUM_SUBCORES, T_C0, CHUNK)
        a = jnp.concatenate(
            [a, jnp.full((NUM_SUBCORES, R_BAL - T_C0, CHUNK), fill,
                         jnp.int32)], axis=1)
        b = flat[cut:].reshape(NUM_SUBCORES, T_C1, CHUNK)
        b = jnp.concatenate(
            [b, jnp.full((NUM_SUBCORES, R_BAL - T_C1, CHUNK), fill,
                         jnp.int32)], axis=1)
        return jnp.concatenate([a, b], axis=0)

    src3 = _bal(src_p, 0)
    dst3 = _bal(dst_p, DUMMY)
    z = jnp.zeros((N, D), jnp.float32)
    ones_chunk = jnp.ones((CHUNK, D), jnp.float32)

    dega, degb = _deg_call(ones_chunk, z, dst3_sym)
    dega = dega[:, :8]
    degb = degb[:, :8]
    q = _tc0(x, W1, dega, degb)
    ra, rb = _edge_call(q, z, src3, dst3)
    q = _tc_mid(ra, rb, dega, degb, b1.reshape(1, D), W2)
    ra, rb = _edge_call(q, z, src3, dst3)
    q = _tc_mid(ra, rb, dega, degb, b2.reshape(1, D), W3)
    ra, rb = _edge_call(q, z, src3, dst3)
    score, issues = _tc_final(
        ra, rb, dega, degb, b3.reshape(1, D),
        Ws1, bs1.reshape(1, -1), Ws2, bs2.reshape(1, -1),
        Wi1, bi1.reshape(1, -1), Wi2, bi2.reshape(1, -1))
    return (score, issues)
